# Initial kernel scaffold; baseline (speedup 1.0000x reference)
import jax
import jax.numpy as jnp
from jax.experimental import pallas as pl


def kernel(expert_indices, table):
    raise NotImplementedError("write your pallas kernel here")


# speedup vs baseline: 1.9444x; 1.9444x over previous
import functools

import jax
import jax.numpy as jnp
from jax import lax
from jax.experimental import pallas as pl
from jax.experimental.pallas import tpu as pltpu
from jax.experimental.pallas import tpu_sc as plsc

_NC = 2
_NS = 16
_NW = _NC * _NS
_CHUNK = 128


def _normalize_body(table_ref, out_ref):
    x = table_ref[...]
    norm = jnp.sqrt(jnp.sum(x * x, axis=1, keepdims=True))
    out_ref[...] = x / jnp.maximum(norm, 1e-12)


def _normalize_table(table):
    return pl.pallas_call(
        _normalize_body,
        out_shape=jax.ShapeDtypeStruct(table.shape, table.dtype),
    )(table)


@functools.cache
def _make_sc_gather(B, D, dtype):
    b_per_w = B // _NW
    n_chunks = b_per_w // _CHUNK
    mesh = plsc.VectorSubcoreMesh(
        core_axis_name="c", subcore_axis_name="s",
        num_cores=_NC, num_subcores=_NS)

    @functools.partial(
        pl.kernel,
        out_type=jax.ShapeDtypeStruct((B, D), dtype),
        mesh=mesh,
        scratch_types=[
            pltpu.VMEM((n_chunks, _CHUNK), jnp.int32),
            pltpu.VMEM((b_per_w, D), dtype),
            pltpu.SemaphoreType.DMA,
        ],
    )
    def gather(table_hbm, idx_hbm, out_hbm, idx_v, rows_v, sem):
        wid = lax.axis_index("s") * _NC + lax.axis_index("c")
        pltpu.sync_copy(idx_hbm.at[wid], idx_v)
        copies = [
            pltpu.async_copy(
                table_hbm.at[idx_v.at[j]],
                rows_v.at[pl.ds(j * _CHUNK, _CHUNK)],
                sem,
            )
            for j in range(n_chunks)
        ]
        for c in copies:
            c.wait()
        pltpu.sync_copy(rows_v, out_hbm.at[pl.ds(wid * b_per_w, b_per_w)])

    return gather


def kernel(expert_indices, table):
    B = expert_indices.shape[0]
    D = table.shape[1]
    table_n = _normalize_table(table)
    idx = expert_indices.astype(jnp.int32).reshape(_NW, B // _NW // _CHUNK, _CHUNK)
    return _make_sc_gather(B, D, table.dtype)(table_n, idx)
